# baseline (speedup 1.0000x reference)
import functools

import jax
import jax.numpy as jnp
from jax import lax
from jax.experimental import pallas as pl
from jax.experimental.pallas import tpu as pltpu
from jax.experimental.pallas import tpu_sc as plsc

N = 10000
E = 320000
D = 128
H = 128
C = 10
G = 128

NC = 2
NS = 16
DH = D // NC
CH = 128
NCHK = E // CH
CPB = 12
NBODY = 13
CPT = CPB * NBODY
DEGHALF = CPT // 2
DEGW = 16
NR = 6
GLAG = 3
NP = 10240
RPT = NP // NS
RQ = 128

BN = 1000
NB = N // BN


def _sc_agg_body(with_deg, *refs):
    if with_deg:
        (src4, dst2, x_hbm, acc_out, deg_out) = refs[:5]
        rest = refs[5:]
    else:
        (src4, dst2, x_hbm, acc_out) = refs[:4]
        rest = refs[4:]
    s0, d0, s1, d1 = rest[:4]
    rowsL = rest[4:4 + NR]
    rest = rest[4 + NR:]
    if with_deg:
        ones, zdeg, acc_sh, deg_sh = rest[:4]
        rest = rest[4:]
    else:
        acc_sh = rest[0]
        rest = rest[1:]
    gsems = rest[:NR]
    ssems = rest[NR:2 * NR]

    c = lax.axis_index("c")
    s = lax.axis_index("s")
    cbase = s * CPT
    rows0 = rowsL[0]

    def zrow(i, _):
        for j in range(DH // 16):
            rows0[i, pl.ds(j * 16, 16)] = jnp.zeros((16,), jnp.float32)
        return 0
    lax.fori_loop(0, CH, zrow, 0)
    if with_deg:
        def orow(i, _):
            ones[i, :] = jnp.ones((16,), jnp.float32)
            return 0
        lax.fori_loop(0, CH, orow, 0)
        def zdrow(i, _):
            zdeg[i, :] = jnp.zeros((16,), jnp.float32)
            return 0
        lax.fori_loop(0, RPT, zdrow, 0)

    for j in range(RPT // RQ):
        pltpu.sync_copy(rows0, acc_sh.at[pl.ds(s * RPT + j * RQ, RQ)])
    if with_deg:
        pltpu.sync_copy(zdeg, deg_sh.at[pl.ds(s * RPT, RPT)])
    plsc.subcore_barrier()

    def sref(p):
        if p < 0:
            return s1.at[6 + p]
        return (s0 if p < 6 else s1).at[p % 6]

    def dref(p):
        if p < 0:
            return d1.at[6 + p]
        return (d0 if p < 6 else d1).at[p % 6]

    def deg_add(t_val, didx):
        if with_deg:
            @pl.when(((c == 0) & (t_val < DEGHALF))
                     | ((c != 0) & (t_val >= DEGHALF)))
            def _():
                pltpu.sync_copy(ones, deg_sh.at[didx], add=True)

    def body(i, _):
        row0 = cbase + i * CPB
        pltpu.sync_copy(src4.at[pl.ds(c * NCHK + row0, 6)], s0)
        pltpu.sync_copy(dst2.at[pl.ds(row0, 6)], d0)
        for q in range(CPB):
            if q == 6:
                pltpu.sync_copy(src4.at[pl.ds(c * NCHK + row0 + 6, 6)], s1)
                pltpu.sync_copy(dst2.at[pl.ds(row0 + 6, 6)], d1)
            r = q % NR
            if q >= NR:
                pltpu.make_async_copy(
                    rowsL[r], acc_sh.at[dref(q - NR)], ssems[r]).wait()
            else:
                @pl.when(i > 0)
                def _(q=q, r=r):
                    pltpu.make_async_copy(
                        rowsL[r], acc_sh.at[dref(q - NR)], ssems[r]).wait()
            pltpu.async_copy(x_hbm.at[sref(q)], rowsL[r], gsems[r])
            t2 = i * CPB + q - GLAG
            r2 = (q - GLAG) % NR
            if q >= GLAG:
                pltpu.make_async_copy(
                    x_hbm.at[sref(q - GLAG)], rowsL[r2], gsems[r2]).wait()
                pltpu.async_copy(rowsL[r2], acc_sh.at[dref(q - GLAG)],
                                 ssems[r2], add=True)
                deg_add(t2, dref(q - GLAG))
            else:
                @pl.when(i > 0)
                def _(q=q, r2=r2, t2=t2):
                    pltpu.make_async_copy(
                        x_hbm.at[sref(q - GLAG)], rowsL[r2],
                        gsems[r2]).wait()
                    pltpu.async_copy(rowsL[r2], acc_sh.at[dref(q - GLAG)],
                                     ssems[r2], add=True)
                    deg_add(t2, dref(q - GLAG))
        return 0
    lax.fori_loop(0, NBODY, body, 0)

    for p in range(CPB - GLAG, CPB):
        r = p % NR
        pltpu.make_async_copy(x_hbm.at[sref(p)], rowsL[r], gsems[r]).wait()
        pltpu.async_copy(rowsL[r], acc_sh.at[dref(p)], ssems[r], add=True)
        deg_add((NBODY - 1) * CPB + p, dref(p))
    for p in range(CPB - NR, CPB):
        r = p % NR
        pltpu.make_async_copy(rowsL[r], acc_sh.at[dref(p)], ssems[r]).wait()

    @pl.when(s < NCHK - NS * CPT)
    def _():
        kx = NS * CPT + s
        pltpu.sync_copy(src4.at[pl.ds(c * NCHK + kx, 1)], s0.at[pl.ds(0, 1)])
        pltpu.sync_copy(dst2.at[pl.ds(kx, 1)], d0.at[pl.ds(0, 1)])
        pltpu.async_copy(x_hbm.at[s0.at[0]], rowsL[0], gsems[0]).wait()
        pltpu.async_copy(rowsL[0], acc_sh.at[d0.at[0]], ssems[0],
                         add=True).wait()
        if with_deg:
            @pl.when((s % 2) == c)
            def _():
                pltpu.sync_copy(ones, deg_sh.at[d0.at[0]], add=True)

    plsc.subcore_barrier()

    for j in range(RPT // RQ):
        r0 = s * RPT + j * RQ
        pltpu.sync_copy(acc_sh.at[pl.ds(r0, RQ)], acc_out.at[c, pl.ds(r0, RQ)])
    if with_deg:
        pltpu.sync_copy(deg_sh.at[pl.ds(s * RPT, RPT)],
                        deg_out.at[c, pl.ds(s * RPT, RPT)])


def _make_sc_agg(with_deg):
    mesh = plsc.VectorSubcoreMesh(core_axis_name="c", subcore_axis_name="s")
    out_type = [jax.ShapeDtypeStruct((NC, NP, DH), jnp.float32)]
    scratch = [
        pltpu.VMEM((6, CH), jnp.int32),
        pltpu.VMEM((6, CH), jnp.int32),
        pltpu.VMEM((6, CH), jnp.int32),
        pltpu.VMEM((6, CH), jnp.int32),
    ]
    scratch.extend([pltpu.VMEM((CH, DH), jnp.float32)] * NR)
    if with_deg:
        out_type.append(jax.ShapeDtypeStruct((NC, NP, DEGW), jnp.float32))
        scratch.append(pltpu.VMEM((CH, DEGW), jnp.float32))
        scratch.append(pltpu.VMEM((RPT, DEGW), jnp.float32))
    scratch.append(pltpu.VMEM_SHARED((NP, DH), jnp.float32))
    if with_deg:
        scratch.append(pltpu.VMEM_SHARED((NP, DEGW), jnp.float32))
    scratch.extend([pltpu.SemaphoreType.DMA] * (2 * NR))
    return pl.kernel(
        functools.partial(_sc_agg_body, with_deg),
        out_type=out_type,
        mesh=mesh,
        scratch_types=scratch,
        compiler_params=pltpu.CompilerParams(use_tc_tiling_on_sc=False),
    )


_sc_agg_deg = _make_sc_agg(True)
_sc_agg = _make_sc_agg(False)


def _dense_body(acc_ref, deg_ref, x_ref, wl_ref, bl_ref, wr_ref, out_ref):
    deg = deg_ref[0, :, 0:1] + deg_ref[1, :, 0:1]
    invd = 1.0 / jnp.maximum(deg, 1.0)
    hL = lax.dot_general(acc_ref[0] * invd, wl_ref[:, :DH],
                         (((1,), (1,)), ((), ())),
                         preferred_element_type=jnp.float32)
    hR = lax.dot_general(acc_ref[1] * invd, wl_ref[:, DH:],
                         (((1,), (1,)), ((), ())),
                         preferred_element_type=jnp.float32)
    h = hL + hR + bl_ref[...]
    h = h + lax.dot_general(x_ref[...], wr_ref[...], (((1,), (1,)), ((), ())),
                            preferred_element_type=jnp.float32)
    out_ref[...] = jnp.maximum(h, 0.0)


_dense = pl.pallas_call(
    _dense_body,
    grid=(NB,),
    in_specs=[
        pl.BlockSpec((NC, BN, DH), lambda i: (0, i, 0)),
        pl.BlockSpec((NC, BN, DEGW), lambda i: (0, i, 0)),
        pl.BlockSpec((BN, D), lambda i: (i, 0)),
        pl.BlockSpec((H, D), lambda i: (0, 0)),
        pl.BlockSpec((1, H), lambda i: (0, 0)),
        pl.BlockSpec((H, D), lambda i: (0, 0)),
    ],
    out_specs=pl.BlockSpec((BN, H), lambda i: (i, 0)),
    out_shape=jax.ShapeDtypeStruct((N, H), jnp.float32),
)


def _dense2_body(acc_ref, deg_ref, h_ref, wl_ref, bl_ref, wr_ref,
                 batch_ref, wfc_ref, bfc_ref, out_ref, pooled, cnts):
    i = pl.program_id(0)

    @pl.when(i == 0)
    def _():
        pooled[...] = jnp.zeros((G, H), jnp.float32)
        cnts[...] = jnp.zeros((G, 128), jnp.float32)

    deg = deg_ref[0, :, 0:1] + deg_ref[1, :, 0:1]
    invd = 1.0 / jnp.maximum(deg, 1.0)
    hL = lax.dot_general(acc_ref[0] * invd, wl_ref[:, :DH],
                         (((1,), (1,)), ((), ())),
                         preferred_element_type=jnp.float32)
    hR = lax.dot_general(acc_ref[1] * invd, wl_ref[:, DH:],
                         (((1,), (1,)), ((), ())),
                         preferred_element_type=jnp.float32)
    h = hL + hR + bl_ref[...]
    h = h + lax.dot_general(h_ref[...], wr_ref[...], (((1,), (1,)), ((), ())),
                            preferred_element_type=jnp.float32)
    h2 = jnp.maximum(h, 0.0)

    bt = batch_ref[0]
    gids = lax.broadcasted_iota(jnp.int32, (G, 1), 0)
    oht = jnp.where(bt == gids, 1.0, 0.0).astype(jnp.float32)
    pooled[...] += lax.dot_general(oht, h2, (((1,), (0,)), ((), ())),
                                   preferred_element_type=jnp.float32)
    cnts[...] += jnp.broadcast_to(
        jnp.sum(oht, axis=1, keepdims=True), (G, 128))

    @pl.when(i == NB - 1)
    def _():
        pm = pooled[...] / jnp.maximum(cnts[:, 0:1], 1.0)
        logits = lax.dot_general(pm, wfc_ref[...], (((1,), (1,)), ((), ())),
                                 preferred_element_type=jnp.float32)
        logits = logits + bfc_ref[...]
        m = jnp.max(logits, axis=-1, keepdims=True)
        ls = logits - m
        out_ref[...] = ls - jnp.log(
            jnp.sum(jnp.exp(ls), axis=-1, keepdims=True))


_dense2 = pl.pallas_call(
    _dense2_body,
    grid=(NB,),
    in_specs=[
        pl.BlockSpec((NC, BN, DH), lambda i: (0, i, 0)),
        pl.BlockSpec((NC, BN, DEGW), lambda i: (0, i, 0)),
        pl.BlockSpec((BN, H), lambda i: (i, 0)),
        pl.BlockSpec((H, H), lambda i: (0, 0)),
        pl.BlockSpec((1, H), lambda i: (0, 0)),
        pl.BlockSpec((H, H), lambda i: (0, 0)),
        pl.BlockSpec((1, 1, BN), lambda i: (i, 0, 0)),
        pl.BlockSpec((128, H), lambda i: (0, 0)),
        pl.BlockSpec((1, 128), lambda i: (0, 0)),
    ],
    out_specs=pl.BlockSpec((G, 128), lambda i: (0, 0)),
    out_shape=jax.ShapeDtypeStruct((G, 128), jnp.float32),
    scratch_shapes=[
        pltpu.VMEM((G, H), jnp.float32),
        pltpu.VMEM((G, 128), jnp.float32),
    ],
    compiler_params=pltpu.CompilerParams(
        dimension_semantics=("arbitrary",)),
)


def kernel(x, edge_index, batch, W1l, b1l, W1r, W2l, b2l, W2r, Wfc, bfc):
    src = edge_index[0]
    dst = edge_index[1]
    src4 = jnp.concatenate([src * 2, src * 2 + 1]).reshape(NC * NCHK, CH)
    dst2 = dst.reshape(NCHK, CH)
    xview = x.reshape(NC * N, DH)

    acc1, deg = _sc_agg_deg(src4, dst2, xview)
    h = _dense(acc1, deg, x, W1l, b1l.reshape(1, H), W1r)

    (acc2,) = _sc_agg(src4, dst2, h.reshape(NC * N, DH))

    batch3 = batch.reshape(NB, 1, BN)
    wfc_pad = jnp.zeros((128, H), jnp.float32).at[:C].set(Wfc)
    bfc_pad = jnp.full((1, 128), -1e30, jnp.float32).at[0, :C].set(bfc)
    out = _dense2(acc2, deg, h, W2l, b2l.reshape(1, H), W2r,
                  batch3, wfc_pad, bfc_pad)
    return out[:, :C]


# speedup vs baseline: 1.0392x; 1.0392x over previous
import functools

import jax
import jax.numpy as jnp
from jax import lax
from jax.experimental import pallas as pl
from jax.experimental.pallas import tpu as pltpu
from jax.experimental.pallas import tpu_sc as plsc

N = 10000
E = 320000
D = 128
H = 128
C = 10
G = 128

NC = 2
NS = 16
DH = D // NC
CH = 128
NCHK = E // CH
CPB = 12
NBODY = 13
CPT = CPB * NBODY
DEGHALF = CPT // 2
DEGW = 16
NR = 6
GLAG = 5
NP = 10240
RPT = NP // NS
RQ = 128

BN = 1000
NB = N // BN


def _sc_agg_body(with_deg, *refs):
    if with_deg:
        (src4, dst2, x_hbm, acc_out, deg_out) = refs[:5]
        rest = refs[5:]
    else:
        (src4, dst2, x_hbm, acc_out) = refs[:4]
        rest = refs[4:]
    s0, d0, s1, d1 = rest[:4]
    rowsL = rest[4:4 + NR]
    rest = rest[4 + NR:]
    if with_deg:
        ones, zdeg, acc_sh, deg_sh = rest[:4]
        rest = rest[4:]
    else:
        acc_sh = rest[0]
        rest = rest[1:]
    gsems = rest[:NR]
    ssems = rest[NR:2 * NR]

    c = lax.axis_index("c")
    s = lax.axis_index("s")
    cbase = s * CPT
    rows0 = rowsL[0]

    def zrow(i, _):
        for j in range(DH // 16):
            rows0[i, pl.ds(j * 16, 16)] = jnp.zeros((16,), jnp.float32)
        return 0
    lax.fori_loop(0, CH, zrow, 0)
    if with_deg:
        def orow(i, _):
            ones[i, :] = jnp.ones((16,), jnp.float32)
            return 0
        lax.fori_loop(0, CH, orow, 0)
        def zdrow(i, _):
            zdeg[i, :] = jnp.zeros((16,), jnp.float32)
            return 0
        lax.fori_loop(0, RPT, zdrow, 0)

    for j in range(RPT // RQ):
        pltpu.sync_copy(rows0, acc_sh.at[pl.ds(s * RPT + j * RQ, RQ)])
    if with_deg:
        pltpu.sync_copy(zdeg, deg_sh.at[pl.ds(s * RPT, RPT)])
    plsc.subcore_barrier()

    def sref(p):
        if p < 0:
            return s1.at[6 + p]
        return (s0 if p < 6 else s1).at[p % 6]

    def dref(p):
        if p < 0:
            return d1.at[6 + p]
        return (d0 if p < 6 else d1).at[p % 6]

    def deg_add(t_val, didx):
        if with_deg:
            @pl.when(((c == 0) & (t_val < DEGHALF))
                     | ((c != 0) & (t_val >= DEGHALF)))
            def _():
                pltpu.sync_copy(ones, deg_sh.at[didx], add=True)

    def body(i, _):
        row0 = cbase + i * CPB
        pltpu.sync_copy(src4.at[pl.ds(c * NCHK + row0, 6)], s0)
        pltpu.sync_copy(dst2.at[pl.ds(row0, 6)], d0)
        for q in range(CPB):
            if q == 6:
                pltpu.sync_copy(src4.at[pl.ds(c * NCHK + row0 + 6, 6)], s1)
                pltpu.sync_copy(dst2.at[pl.ds(row0 + 6, 6)], d1)
            r = q % NR
            if q >= NR:
                pltpu.make_async_copy(
                    rowsL[r], acc_sh.at[dref(q - NR)], ssems[r]).wait()
            else:
                @pl.when(i > 0)
                def _(q=q, r=r):
                    pltpu.make_async_copy(
                        rowsL[r], acc_sh.at[dref(q - NR)], ssems[r]).wait()
            pltpu.async_copy(x_hbm.at[sref(q)], rowsL[r], gsems[r])
            t2 = i * CPB + q - GLAG
            r2 = (q - GLAG) % NR
            if q >= GLAG:
                pltpu.make_async_copy(
                    x_hbm.at[sref(q - GLAG)], rowsL[r2], gsems[r2]).wait()
                pltpu.async_copy(rowsL[r2], acc_sh.at[dref(q - GLAG)],
                                 ssems[r2], add=True)
                deg_add(t2, dref(q - GLAG))
            else:
                @pl.when(i > 0)
                def _(q=q, r2=r2, t2=t2):
                    pltpu.make_async_copy(
                        x_hbm.at[sref(q - GLAG)], rowsL[r2],
                        gsems[r2]).wait()
                    pltpu.async_copy(rowsL[r2], acc_sh.at[dref(q - GLAG)],
                                     ssems[r2], add=True)
                    deg_add(t2, dref(q - GLAG))
        return 0
    lax.fori_loop(0, NBODY, body, 0)

    for p in range(CPB - GLAG, CPB):
        r = p % NR
        pltpu.make_async_copy(x_hbm.at[sref(p)], rowsL[r], gsems[r]).wait()
        pltpu.async_copy(rowsL[r], acc_sh.at[dref(p)], ssems[r], add=True)
        deg_add((NBODY - 1) * CPB + p, dref(p))
    for p in range(CPB - NR, CPB):
        r = p % NR
        pltpu.make_async_copy(rowsL[r], acc_sh.at[dref(p)], ssems[r]).wait()

    @pl.when(s < NCHK - NS * CPT)
    def _():
        kx = NS * CPT + s
        pltpu.sync_copy(src4.at[pl.ds(c * NCHK + kx, 1)], s0.at[pl.ds(0, 1)])
        pltpu.sync_copy(dst2.at[pl.ds(kx, 1)], d0.at[pl.ds(0, 1)])
        pltpu.async_copy(x_hbm.at[s0.at[0]], rowsL[0], gsems[0]).wait()
        pltpu.async_copy(rowsL[0], acc_sh.at[d0.at[0]], ssems[0],
                         add=True).wait()
        if with_deg:
            @pl.when((s % 2) == c)
            def _():
                pltpu.sync_copy(ones, deg_sh.at[d0.at[0]], add=True)

    plsc.subcore_barrier()

    for j in range(RPT // RQ):
        r0 = s * RPT + j * RQ
        pltpu.sync_copy(acc_sh.at[pl.ds(r0, RQ)], acc_out.at[c, pl.ds(r0, RQ)])
    if with_deg:
        pltpu.sync_copy(deg_sh.at[pl.ds(s * RPT, RPT)],
                        deg_out.at[c, pl.ds(s * RPT, RPT)])


def _make_sc_agg(with_deg):
    mesh = plsc.VectorSubcoreMesh(core_axis_name="c", subcore_axis_name="s")
    out_type = [jax.ShapeDtypeStruct((NC, NP, DH), jnp.float32)]
    scratch = [
        pltpu.VMEM((6, CH), jnp.int32),
        pltpu.VMEM((6, CH), jnp.int32),
        pltpu.VMEM((6, CH), jnp.int32),
        pltpu.VMEM((6, CH), jnp.int32),
    ]
    scratch.extend([pltpu.VMEM((CH, DH), jnp.float32)] * NR)
    if with_deg:
        out_type.append(jax.ShapeDtypeStruct((NC, NP, DEGW), jnp.float32))
        scratch.append(pltpu.VMEM((CH, DEGW), jnp.float32))
        scratch.append(pltpu.VMEM((RPT, DEGW), jnp.float32))
    scratch.append(pltpu.VMEM_SHARED((NP, DH), jnp.float32))
    if with_deg:
        scratch.append(pltpu.VMEM_SHARED((NP, DEGW), jnp.float32))
    scratch.extend([pltpu.SemaphoreType.DMA] * (2 * NR))
    return pl.kernel(
        functools.partial(_sc_agg_body, with_deg),
        out_type=out_type,
        mesh=mesh,
        scratch_types=scratch,
        compiler_params=pltpu.CompilerParams(use_tc_tiling_on_sc=False),
    )


_sc_agg_deg = _make_sc_agg(True)
_sc_agg = _make_sc_agg(False)


def _dense_body(acc_ref, deg_ref, x_ref, wl_ref, bl_ref, wr_ref, out_ref):
    deg = deg_ref[0, :, 0:1] + deg_ref[1, :, 0:1]
    invd = 1.0 / jnp.maximum(deg, 1.0)
    hL = lax.dot_general(acc_ref[0] * invd, wl_ref[:, :DH],
                         (((1,), (1,)), ((), ())),
                         preferred_element_type=jnp.float32)
    hR = lax.dot_general(acc_ref[1] * invd, wl_ref[:, DH:],
                         (((1,), (1,)), ((), ())),
                         preferred_element_type=jnp.float32)
    h = hL + hR + bl_ref[...]
    h = h + lax.dot_general(x_ref[...], wr_ref[...], (((1,), (1,)), ((), ())),
                            preferred_element_type=jnp.float32)
    out_ref[...] = jnp.maximum(h, 0.0)


_dense = pl.pallas_call(
    _dense_body,
    grid=(NB,),
    in_specs=[
        pl.BlockSpec((NC, BN, DH), lambda i: (0, i, 0)),
        pl.BlockSpec((NC, BN, DEGW), lambda i: (0, i, 0)),
        pl.BlockSpec((BN, D), lambda i: (i, 0)),
        pl.BlockSpec((H, D), lambda i: (0, 0)),
        pl.BlockSpec((1, H), lambda i: (0, 0)),
        pl.BlockSpec((H, D), lambda i: (0, 0)),
    ],
    out_specs=pl.BlockSpec((BN, H), lambda i: (i, 0)),
    out_shape=jax.ShapeDtypeStruct((N, H), jnp.float32),
)


def _dense2_body(acc_ref, deg_ref, h_ref, wl_ref, bl_ref, wr_ref,
                 batch_ref, wfc_ref, bfc_ref, out_ref, pooled, cnts):
    i = pl.program_id(0)

    @pl.when(i == 0)
    def _():
        pooled[...] = jnp.zeros((G, H), jnp.float32)
        cnts[...] = jnp.zeros((G, 128), jnp.float32)

    deg = deg_ref[0, :, 0:1] + deg_ref[1, :, 0:1]
    invd = 1.0 / jnp.maximum(deg, 1.0)
    hL = lax.dot_general(acc_ref[0] * invd, wl_ref[:, :DH],
                         (((1,), (1,)), ((), ())),
                         preferred_element_type=jnp.float32)
    hR = lax.dot_general(acc_ref[1] * invd, wl_ref[:, DH:],
                         (((1,), (1,)), ((), ())),
                         preferred_element_type=jnp.float32)
    h = hL + hR + bl_ref[...]
    h = h + lax.dot_general(h_ref[...], wr_ref[...], (((1,), (1,)), ((), ())),
                            preferred_element_type=jnp.float32)
    h2 = jnp.maximum(h, 0.0)

    bt = batch_ref[0]
    gids = lax.broadcasted_iota(jnp.int32, (G, 1), 0)
    oht = jnp.where(bt == gids, 1.0, 0.0).astype(jnp.float32)
    pooled[...] += lax.dot_general(oht, h2, (((1,), (0,)), ((), ())),
                                   preferred_element_type=jnp.float32)
    cnts[...] += jnp.broadcast_to(
        jnp.sum(oht, axis=1, keepdims=True), (G, 128))

    @pl.when(i == NB - 1)
    def _():
        pm = pooled[...] / jnp.maximum(cnts[:, 0:1], 1.0)
        logits = lax.dot_general(pm, wfc_ref[...], (((1,), (1,)), ((), ())),
                                 preferred_element_type=jnp.float32)
        logits = logits + bfc_ref[...]
        m = jnp.max(logits, axis=-1, keepdims=True)
        ls = logits - m
        out_ref[...] = ls - jnp.log(
            jnp.sum(jnp.exp(ls), axis=-1, keepdims=True))


_dense2 = pl.pallas_call(
    _dense2_body,
    grid=(NB,),
    in_specs=[
        pl.BlockSpec((NC, BN, DH), lambda i: (0, i, 0)),
        pl.BlockSpec((NC, BN, DEGW), lambda i: (0, i, 0)),
        pl.BlockSpec((BN, H), lambda i: (i, 0)),
        pl.BlockSpec((H, H), lambda i: (0, 0)),
        pl.BlockSpec((1, H), lambda i: (0, 0)),
        pl.BlockSpec((H, H), lambda i: (0, 0)),
        pl.BlockSpec((1, 1, BN), lambda i: (i, 0, 0)),
        pl.BlockSpec((128, H), lambda i: (0, 0)),
        pl.BlockSpec((1, 128), lambda i: (0, 0)),
    ],
    out_specs=pl.BlockSpec((G, 128), lambda i: (0, 0)),
    out_shape=jax.ShapeDtypeStruct((G, 128), jnp.float32),
    scratch_shapes=[
        pltpu.VMEM((G, H), jnp.float32),
        pltpu.VMEM((G, 128), jnp.float32),
    ],
    compiler_params=pltpu.CompilerParams(
        dimension_semantics=("arbitrary",)),
)


def kernel(x, edge_index, batch, W1l, b1l, W1r, W2l, b2l, W2r, Wfc, bfc):
    src = edge_index[0]
    dst = edge_index[1]
    src4 = jnp.concatenate([src * 2, src * 2 + 1]).reshape(NC * NCHK, CH)
    dst2 = dst.reshape(NCHK, CH)
    xview = x.reshape(NC * N, DH)

    acc1, deg = _sc_agg_deg(src4, dst2, xview)
    h = _dense(acc1, deg, x, W1l, b1l.reshape(1, H), W1r)

    (acc2,) = _sc_agg(src4, dst2, h.reshape(NC * N, DH))

    batch3 = batch.reshape(NB, 1, BN)
    wfc_pad = jnp.zeros((128, H), jnp.float32).at[:C].set(Wfc)
    bfc_pad = jnp.full((1, 128), -1e30, jnp.float32).at[0, :C].set(bfc)
    out = _dense2(acc2, deg, h, W2l, b2l.reshape(1, H), W2r,
                  batch3, wfc_pad, bfc_pad)
    return out[:, :C]
